# baseline (speedup 1.0000x reference)
import jax
import jax.numpy as jnp
from jax.experimental import pallas as pl

NSAMPLE = 32


def _copy_kernel(x_ref, o_ref):
    o_ref[...] = x_ref[...]


def _group(feat, idx):
    B, C, N = feat.shape
    _, P, S = idx.shape
    idx_flat = idx.reshape(B, 1, P * S)
    gathered = jnp.take_along_axis(feat, jnp.broadcast_to(idx_flat, (B, C, P * S)), axis=2)
    return gathered.reshape(B, C, P, S)


def kernel(xyz, new_xyz, features):
    dists = jnp.sum((new_xyz[:, :, None, :] - xyz[:, None, :, :]) ** 2, axis=-1)
    _, idx = jax.lax.top_k(-dists, NSAMPLE)
    idx = idx.astype(jnp.int32)

    xyz_trans = jnp.transpose(xyz, (0, 2, 1))
    grouped_xyz = _group(xyz_trans, idx)
    grouped_xyz = grouped_xyz - jnp.transpose(new_xyz, (0, 2, 1))[:, :, :, None]

    grouped_features = _group(features, idx)
    new_features = jnp.concatenate([grouped_xyz, grouped_features], axis=1)
    B, C, P, S = new_features.shape
    out = pl.pallas_call(
        _copy_kernel,
        grid=(B, P // 256),
        in_specs=[pl.BlockSpec((1, C, 256, S), lambda b, p: (b, 0, p, 0))],
        out_specs=pl.BlockSpec((1, C, 256, S), lambda b, p: (b, 0, p, 0)),
        out_shape=jax.ShapeDtypeStruct(new_features.shape, new_features.dtype),
    )(new_features)
    return out


# speedup vs baseline: 96.4485x; 96.4485x over previous
import functools

import jax
import jax.numpy as jnp
from jax import lax
from jax.experimental import pallas as pl
from jax.experimental.pallas import tpu as pltpu
from jax.experimental.pallas import tpu_sc as plsc

K = 32
N = 8192
P = 2048
B = 4
C = 64
PB = 128
NW = 32
CHUNKS = NW // B
QPW = P // CHUNKS
FLAT = QPW * K


def _topk_kernel(new_ref, xyz_ref, idx_ref):
    d = None
    for c in range(3):
        q = new_ref[0, :, c].reshape(PB, 1)
        k = xyz_ref[0, c, :].reshape(1, N)
        sq = (q - k) ** 2
        d = sq if d is None else d + sq
    iota = lax.broadcasted_iota(jnp.int32, (PB, N), 1)
    work = d
    cols = []
    for _ in range(K):
        m = jnp.min(work, axis=1, keepdims=True)
        am = jnp.min(jnp.where(work == m, iota, N), axis=1, keepdims=True)
        cols.append(am)
        work = jnp.where(iota == am, jnp.float32(jnp.inf), work)
    idx_ref[0] = jnp.concatenate(cols, axis=1)


def _grouping_kernel(xyz_hbm, feat_hbm, newq_hbm, idx_hbm, qidx_hbm, out_hbm,
                     idx_v, qidx_v, table_v, newq_v, out_v):
    cid = lax.axis_index("c")
    sid = lax.axis_index("s")
    wid = sid * 2 + cid
    b = wid // CHUNKS
    chunk = wid % CHUNKS
    base = chunk * FLAT
    pbase = chunk * QPW

    pltpu.sync_copy(idx_hbm.at[pl.ds(b * P * K + base, FLAT)], idx_v)
    pltpu.sync_copy(qidx_hbm, qidx_v)

    def gather_all(i, _):
        iv = idx_v[pl.ds(i * 16, 16)]
        out_v[pl.ds(i * 16, 16)] = plsc.load_gather(table_v, [iv])
        return 0

    def gather_sub(i, _):
        iv = idx_v[pl.ds(i * 16, 16)]
        g = plsc.load_gather(table_v, [iv])
        qv = qidx_v[pl.ds(i * 16, 16)]
        g2 = plsc.load_gather(newq_v, [qv])
        out_v[pl.ds(i * 16, 16)] = g - g2
        return 0

    for ch in range(3):
        pltpu.sync_copy(xyz_hbm.at[pl.ds((b * 3 + ch) * N, N)], table_v)
        pltpu.sync_copy(newq_hbm.at[pl.ds((b * 3 + ch) * P + pbase, QPW)], newq_v)
        lax.fori_loop(0, FLAT // 16, gather_sub, 0)
        pltpu.sync_copy(out_v, out_hbm.at[pl.ds((b * (3 + C) + ch) * P * K + base, FLAT)])

    def feat_body(ch, _):
        pltpu.sync_copy(feat_hbm.at[pl.ds((b * C + ch) * N, N)], table_v)
        lax.fori_loop(0, FLAT // 16, gather_all, 0)
        pltpu.sync_copy(
            out_v, out_hbm.at[pl.ds((b * (3 + C) + 3 + ch) * P * K + base, FLAT)])
        return 0

    lax.fori_loop(0, C, feat_body, 0)


def kernel(xyz, new_xyz, features):
    xyz8 = jnp.zeros((B, 8, N), jnp.float32).at[:, 0:3, :].set(
        jnp.transpose(xyz, (0, 2, 1)))
    new8 = jnp.zeros((B, P, 8), jnp.float32).at[:, :, 0:3].set(new_xyz)
    newq = jnp.transpose(new_xyz, (0, 2, 1))

    idx = pl.pallas_call(
        _topk_kernel,
        grid=(B, P // PB),
        in_specs=[
            pl.BlockSpec((1, PB, 8), lambda b, p: (b, p, 0)),
            pl.BlockSpec((1, 8, N), lambda b, p: (b, 0, 0)),
        ],
        out_specs=pl.BlockSpec((1, PB, K), lambda b, p: (b, p, 0)),
        out_shape=jax.ShapeDtypeStruct((B, P, K), jnp.int32),
    )(new8, xyz8)

    idx_flat = idx.reshape(B * P * K)
    qidx = (jnp.arange(FLAT, dtype=jnp.int32) // K)
    xyz_flat = jnp.transpose(xyz, (0, 2, 1)).reshape(B * 3 * N)
    feat_flat = features.reshape(B * C * N)
    newq_flat = newq.reshape(B * 3 * P)

    mesh = plsc.VectorSubcoreMesh(
        core_axis_name="c", subcore_axis_name="s", num_cores=2, num_subcores=16)
    grouped = pl.kernel(
        _grouping_kernel,
        out_type=jax.ShapeDtypeStruct((B * (3 + C) * P * K,), jnp.float32),
        mesh=mesh,
        compiler_params=pltpu.CompilerParams(needs_layout_passes=False),
        scratch_types=[
            pltpu.VMEM((FLAT,), jnp.int32),
            pltpu.VMEM((FLAT,), jnp.int32),
            pltpu.VMEM((N,), jnp.float32),
            pltpu.VMEM((QPW,), jnp.float32),
            pltpu.VMEM((FLAT,), jnp.float32),
        ],
    )(xyz_flat, feat_flat, newq_flat, idx_flat, qidx)

    return grouped.reshape(B, 3 + C, P, K)


# speedup vs baseline: 137.7945x; 1.4287x over previous
import functools

import jax
import jax.numpy as jnp
from jax import lax
from jax.experimental import pallas as pl
from jax.experimental.pallas import tpu as pltpu
from jax.experimental.pallas import tpu_sc as plsc

K = 32
N = 8192
P = 2048
B = 4
C = 64
PB = 128
NW = 32
CHUNKS = NW // B
QPW = P // CHUNKS
FLAT = QPW * K


def _topk_kernel(new_ref, xyz_ref, idx_ref):
    d = None
    for c in range(3):
        q = new_ref[0, :, c].reshape(PB, 1)
        k = xyz_ref[0, c, :].reshape(1, N)
        sq = (q - k) ** 2
        d = sq if d is None else d + sq
    iota = lax.broadcasted_iota(jnp.int32, (PB, N), 1).astype(jnp.float32)
    work = d
    cols = []
    for _ in range(K):
        m = jnp.min(work, axis=1, keepdims=True)
        am = jnp.min(jnp.where(work == m, iota, jnp.float32(N)), axis=1,
                     keepdims=True)
        cols.append(am)
        work = jnp.where(iota == am, jnp.float32(jnp.inf), work)
    idx_ref[0] = jnp.concatenate(cols, axis=1).astype(jnp.int32)


def _grouping_kernel(xyz_hbm, feat_hbm, newq_hbm, idx_hbm, qidx_hbm, out_hbm,
                     idx_v, qidx_v, table_v, newq_v, out_v):
    cid = lax.axis_index("c")
    sid = lax.axis_index("s")
    wid = sid * 2 + cid
    b = wid // CHUNKS
    chunk = wid % CHUNKS
    base = chunk * FLAT
    pbase = chunk * QPW

    pltpu.sync_copy(idx_hbm.at[pl.ds(b * P * K + base, FLAT)], idx_v)
    pltpu.sync_copy(qidx_hbm, qidx_v)

    def gather_all(i, _):
        iv = idx_v[pl.ds(i * 16, 16)]
        out_v[pl.ds(i * 16, 16)] = plsc.load_gather(table_v, [iv])
        return 0

    def gather_sub(i, _):
        iv = idx_v[pl.ds(i * 16, 16)]
        g = plsc.load_gather(table_v, [iv])
        qv = qidx_v[pl.ds(i * 16, 16)]
        g2 = plsc.load_gather(newq_v, [qv])
        out_v[pl.ds(i * 16, 16)] = g - g2
        return 0

    for ch in range(3):
        pltpu.sync_copy(xyz_hbm.at[pl.ds((b * 3 + ch) * N, N)], table_v)
        pltpu.sync_copy(newq_hbm.at[pl.ds((b * 3 + ch) * P + pbase, QPW)], newq_v)
        lax.fori_loop(0, FLAT // 16, gather_sub, 0)
        pltpu.sync_copy(out_v, out_hbm.at[pl.ds((b * (3 + C) + ch) * P * K + base, FLAT)])

    def feat_body(ch, _):
        pltpu.sync_copy(feat_hbm.at[pl.ds((b * C + ch) * N, N)], table_v)
        lax.fori_loop(0, FLAT // 16, gather_all, 0)
        pltpu.sync_copy(
            out_v, out_hbm.at[pl.ds((b * (3 + C) + 3 + ch) * P * K + base, FLAT)])
        return 0

    lax.fori_loop(0, C, feat_body, 0)


def kernel(xyz, new_xyz, features):
    xyz8 = jnp.zeros((B, 8, N), jnp.float32).at[:, 0:3, :].set(
        jnp.transpose(xyz, (0, 2, 1)))
    new8 = jnp.zeros((B, P, 8), jnp.float32).at[:, :, 0:3].set(new_xyz)
    newq = jnp.transpose(new_xyz, (0, 2, 1))

    idx = pl.pallas_call(
        _topk_kernel,
        grid=(B, P // PB),
        in_specs=[
            pl.BlockSpec((1, PB, 8), lambda b, p: (b, p, 0)),
            pl.BlockSpec((1, 8, N), lambda b, p: (b, 0, 0)),
        ],
        out_specs=pl.BlockSpec((1, PB, K), lambda b, p: (b, p, 0)),
        out_shape=jax.ShapeDtypeStruct((B, P, K), jnp.int32),
    )(new8, xyz8)

    idx_flat = idx.reshape(B * P * K)
    qidx = (jnp.arange(FLAT, dtype=jnp.int32) // K)
    xyz_flat = jnp.transpose(xyz, (0, 2, 1)).reshape(B * 3 * N)
    feat_flat = features.reshape(B * C * N)
    newq_flat = newq.reshape(B * 3 * P)

    mesh = plsc.VectorSubcoreMesh(
        core_axis_name="c", subcore_axis_name="s", num_cores=2, num_subcores=16)
    grouped = pl.kernel(
        _grouping_kernel,
        out_type=jax.ShapeDtypeStruct((B * (3 + C) * P * K,), jnp.float32),
        mesh=mesh,
        compiler_params=pltpu.CompilerParams(needs_layout_passes=False),
        scratch_types=[
            pltpu.VMEM((FLAT,), jnp.int32),
            pltpu.VMEM((FLAT,), jnp.int32),
            pltpu.VMEM((N,), jnp.float32),
            pltpu.VMEM((QPW,), jnp.float32),
            pltpu.VMEM((FLAT,), jnp.float32),
        ],
    )(xyz_flat, feat_flat, newq_flat, idx_flat, qidx)

    return grouped.reshape(B, 3 + C, P, K)
